# baseline (speedup 1.0000x reference)
import functools

import jax
import jax.numpy as jnp
from jax import lax
from jax.experimental import pallas as pl
from jax.experimental.pallas import tpu as pltpu
from jax.experimental.pallas import tpu_sc as plsc

NC, NS = 2, 16
NW = NC * NS
CHUNK = 128
B = 4096 * 200
PW = B // NW
NCHUNK = PW // CHUNK
D = 64

_mesh = plsc.VectorSubcoreMesh(
    core_axis_name="c", subcore_axis_name="s", num_cores=NC, num_subcores=NS
)


@functools.partial(
    pl.kernel,
    out_type=(
        jax.ShapeDtypeStruct((B, D), jnp.float32),
        jax.ShapeDtypeStruct((B, D), jnp.float32),
    ),
    mesh=_mesh,
    compiler_params=pltpu.CompilerParams(use_tc_tiling_on_sc=False),
    scratch_types=[
        pltpu.VMEM((NCHUNK, CHUNK), jnp.int32),
        pltpu.VMEM((NCHUNK, CHUNK), jnp.int32),
        pltpu.VMEM((CHUNK, D), jnp.float32),
        pltpu.VMEM((CHUNK, D), jnp.float32),
        pltpu.SemaphoreType.DMA,
        pltpu.SemaphoreType.DMA,
    ],
)
def _emb_kernel(hour_idx, day_idx, hour_tab, day_tab, out_hour, out_day,
                idx_h, idx_d, rows_h, rows_d, sem_h, sem_d):
    wid = lax.axis_index("s") * NC + lax.axis_index("c")
    pltpu.sync_copy(hour_idx.at[wid], idx_h)
    pltpu.sync_copy(day_idx.at[wid], idx_d)
    base = wid * PW

    def chunk(g, carry):
        off = base + g * CHUNK
        ch = pltpu.async_copy(hour_tab.at[idx_h.at[g]], rows_h, sem_h)
        cd = pltpu.async_copy(day_tab.at[idx_d.at[g]], rows_d, sem_d)
        ch.wait()
        pltpu.sync_copy(rows_h, out_hour.at[pl.ds(off, CHUNK)])
        cd.wait()
        pltpu.sync_copy(rows_d, out_day.at[pl.ds(off, CHUNK)])
        return carry

    lax.fori_loop(0, NCHUNK, chunk, 0)


def kernel(t_hour, t_day, time_in_day_table, day_in_week_table):
    S, T = t_hour.shape
    h = t_hour.astype(jnp.int32).reshape(NW, NCHUNK, CHUNK)
    d = t_day.astype(jnp.int32).reshape(NW, NCHUNK, CHUNK)
    oh, od = _emb_kernel(h, d, time_in_day_table, day_in_week_table)
    return oh.reshape(S, T, D), od.reshape(S, T, D)
